# Initial kernel scaffold; baseline (speedup 1.0000x reference)
import jax
import jax.numpy as jnp
from jax.experimental import pallas as pl


def kernel(node_tokens, edge_index, batch_ids, emb, Ws, bs, gammas, betas, Wp, bp):
    raise NotImplementedError("write your pallas kernel here")


# speedup vs baseline: 5.9031x; 5.9031x over previous
import functools

import jax
import jax.numpy as jnp
from jax import lax
from jax.experimental import pallas as pl
from jax.experimental.pallas import tpu as pltpu
from jax.experimental.pallas import tpu_sc as plsc

N = 10000
E = 320000
H = 128
G = 512
NLAYERS = 3

NC = 2
NS = 16
CHUNK = 128
NWORK = NC * NS
CPW = 80
EP = NWORK * CPW * CHUNK
NPAD = 10112
ROWS_PT = NPAD // NS
OUT_MAIN = 632
OUT_TAIL = N - (NS - 1) * OUT_MAIN

_MESH = plsc.VectorSubcoreMesh(
    core_axis_name="c", subcore_axis_name="s", num_cores=NC, num_subcores=NS
)


@functools.partial(
    pl.kernel,
    out_type=jax.ShapeDtypeStruct((NC, N, H), jnp.float32),
    mesh=_MESH,
    scratch_types=[
        pltpu.VMEM_SHARED((NPAD, H), jnp.float32),
        pltpu.VMEM((CPW, CHUNK), jnp.int32),
        pltpu.VMEM((CHUNK, H), jnp.float32),
    ],
)
def _deg_kernel(dstm_hbm, zeros_hbm, ones_hbm, out_hbm, deg_sh, dstv, ones_v):
    c = lax.axis_index("c")
    t = lax.axis_index("s")
    w = c * NS + t
    pltpu.sync_copy(dstm_hbm.at[pl.ds(w * CPW, CPW)], dstv)
    pltpu.sync_copy(ones_hbm, ones_v)
    pltpu.sync_copy(
        zeros_hbm.at[pl.ds(t * ROWS_PT, ROWS_PT)],
        deg_sh.at[pl.ds(t * ROWS_PT, ROWS_PT)],
    )
    plsc.subcore_barrier()

    def body(j, carry):
        pltpu.sync_copy(ones_v, deg_sh.at[dstv.at[j]], add=True)
        return carry

    lax.fori_loop(0, CPW, body, 0)
    plsc.subcore_barrier()

    @pl.when(t < NS - 1)
    def _copy_main():
        pltpu.sync_copy(
            deg_sh.at[pl.ds(t * OUT_MAIN, OUT_MAIN)],
            out_hbm.at[c, pl.ds(t * OUT_MAIN, OUT_MAIN)],
        )

    @pl.when(t == NS - 1)
    def _copy_tail():
        pltpu.sync_copy(
            deg_sh.at[pl.ds((NS - 1) * OUT_MAIN, OUT_TAIL)],
            out_hbm.at[c, pl.ds((NS - 1) * OUT_MAIN, OUT_TAIL)],
        )


@functools.partial(
    pl.kernel,
    out_type=jax.ShapeDtypeStruct((NC, N, H), jnp.float32),
    mesh=_MESH,
    scratch_types=[
        pltpu.VMEM_SHARED((NPAD, H), jnp.float32),
        pltpu.VMEM((CPW, CHUNK), jnp.int32),
        pltpu.VMEM((CPW, CHUNK), jnp.int32),
        pltpu.VMEM((CHUNK, H), jnp.float32),
        pltpu.SemaphoreType.DMA,
    ],
)
def _mp_kernel(hp_hbm, srcm_hbm, dstm_hbm, zeros_hbm, out_hbm,
               acc_sh, srcv, dstv, rows_v, sem):
    c = lax.axis_index("c")
    t = lax.axis_index("s")
    w = c * NS + t
    pltpu.sync_copy(srcm_hbm.at[pl.ds(w * CPW, CPW)], srcv)
    pltpu.sync_copy(dstm_hbm.at[pl.ds(w * CPW, CPW)], dstv)
    pltpu.sync_copy(
        zeros_hbm.at[pl.ds(t * ROWS_PT, ROWS_PT)],
        acc_sh.at[pl.ds(t * ROWS_PT, ROWS_PT)],
    )
    plsc.subcore_barrier()

    def body(j, carry):
        pltpu.async_copy(hp_hbm.at[srcv.at[j]], rows_v, sem).wait()
        pltpu.sync_copy(rows_v, acc_sh.at[dstv.at[j]], add=True)
        return carry

    lax.fori_loop(0, CPW, body, 0)
    plsc.subcore_barrier()

    @pl.when(t < NS - 1)
    def _copy_main():
        pltpu.sync_copy(
            acc_sh.at[pl.ds(t * OUT_MAIN, OUT_MAIN)],
            out_hbm.at[c, pl.ds(t * OUT_MAIN, OUT_MAIN)],
        )

    @pl.when(t == NS - 1)
    def _copy_tail():
        pltpu.sync_copy(
            acc_sh.at[pl.ds((NS - 1) * OUT_MAIN, OUT_TAIL)],
            out_hbm.at[c, pl.ds((NS - 1) * OUT_MAIN, OUT_TAIL)],
        )


def _embed_body(tok_ref, emb_ref, w_ref, degp_ref, hp_ref, dinv_ref):
    tok = tok_ref[...]
    oh = (tok == lax.broadcasted_iota(jnp.int32, (N, 16), 1)).astype(jnp.float32)
    ew = jnp.dot(emb_ref[...], w_ref[...], preferred_element_type=jnp.float32)
    x0w = jnp.dot(oh, ew, preferred_element_type=jnp.float32)
    degp = degp_ref[...]
    deg = degp[0, :, 0:1] + degp[1, :, 0:1] + 1.0
    dinv = lax.rsqrt(deg)
    dinv_ref[...] = jnp.broadcast_to(dinv, (N, 8))
    hp_ref[...] = x0w * dinv


def _mid_body(accp_ref, hp_ref, dinv_ref, b_ref, g_ref, be_ref, w_ref, out_ref):
    acc = accp_ref[0] + accp_ref[1]
    hp = hp_ref[...]
    dinv = dinv_ref[:, 0:1]
    agg = dinv * (acc + hp) + b_ref[...]
    m = jnp.mean(agg, axis=0, keepdims=True)
    xc = agg - m
    v = jnp.mean(xc * xc, axis=0, keepdims=True)
    xn = xc * lax.rsqrt(v + 1e-5)
    x = jnp.maximum(g_ref[...] * xn + be_ref[...], 0.0)
    out_ref[...] = jnp.dot(x, w_ref[...], preferred_element_type=jnp.float32) * dinv


def _final_body(accp_ref, hp_ref, dinv_ref, b_ref, g_ref, be_ref,
                br_ref, wp_ref, bp_ref, out_ref):
    acc = accp_ref[0] + accp_ref[1]
    hp = hp_ref[...]
    dinv = dinv_ref[:, 0:1]
    agg = dinv * (acc + hp) + b_ref[...]
    m = jnp.mean(agg, axis=0, keepdims=True)
    xc = agg - m
    v = jnp.mean(xc * xc, axis=0, keepdims=True)
    xn = xc * lax.rsqrt(v + 1e-5)
    x = jnp.maximum(g_ref[...] * xn + be_ref[...], 0.0)

    nchunk = 2000
    pooled = jnp.zeros((G, H), jnp.float32)
    cnt = jnp.zeros((G, 1), jnp.float32)
    giota = lax.broadcasted_iota(jnp.int32, (G, nchunk), 0)
    for kc in range(N // nchunk):
        bid = br_ref[0:1, kc * nchunk:(kc + 1) * nchunk]
        ohT = (giota == bid).astype(jnp.float32)
        pooled = pooled + jnp.dot(
            ohT, x[kc * nchunk:(kc + 1) * nchunk],
            preferred_element_type=jnp.float32)
        cnt = cnt + jnp.sum(ohT, axis=1, keepdims=True)
    pooled = pooled / jnp.maximum(cnt, 1.0)
    out_ref[...] = (
        jnp.dot(pooled, wp_ref[...], preferred_element_type=jnp.float32)
        + bp_ref[...]
    )


def _tc(body, out_shape, *args):
    return pl.pallas_call(body, out_shape=out_shape)(*args)


def kernel(node_tokens, edge_index, batch_ids, emb, Ws, bs, gammas, betas, Wp, bp):
    src = edge_index[0].astype(jnp.int32)
    dst = edge_index[1].astype(jnp.int32)
    pad = EP - E
    srcm = jnp.concatenate([src, jnp.zeros((pad,), jnp.int32)]).reshape(EP // CHUNK, CHUNK)
    dstm = jnp.concatenate([dst, jnp.full((pad,), N, jnp.int32)]).reshape(EP // CHUNK, CHUNK)

    zerosH = jnp.zeros((NPAD, H), jnp.float32)
    onesH = jnp.ones((CHUNK, H), jnp.float32)

    degp = _deg_kernel(dstm, zerosH, onesH)

    tok2 = node_tokens.astype(jnp.int32).reshape(N, 1)
    embp = jnp.pad(emb, ((0, 16 - emb.shape[0]), (0, 0)))

    hp, dinv8 = _tc(_embed_body,
                    (jax.ShapeDtypeStruct((N, H), jnp.float32),
                     jax.ShapeDtypeStruct((N, 8), jnp.float32)),
                    tok2, embp, Ws[0], degp)

    for l in range(NLAYERS - 1):
        accp = _mp_kernel(hp, srcm, dstm, zerosH)
        hp = _tc(_mid_body, jax.ShapeDtypeStruct((N, H), jnp.float32),
                 accp, hp, dinv8,
                 bs[l].reshape(1, H), gammas[l].reshape(1, H),
                 betas[l].reshape(1, H), Ws[l + 1])

    accp = _mp_kernel(hp, srcm, dstm, zerosH)
    batch_row = batch_ids.astype(jnp.int32).reshape(1, N)
    l = NLAYERS - 1
    out = _tc(_final_body, jax.ShapeDtypeStruct((G, H), jnp.float32),
              accp, hp, dinv8,
              bs[l].reshape(1, H), gammas[l].reshape(1, H),
              betas[l].reshape(1, H), batch_row, Wp, bp.reshape(1, H))
    return out


# speedup vs baseline: 6.1005x; 1.0334x over previous
import functools

import jax
import jax.numpy as jnp
from jax import lax
from jax.experimental import pallas as pl
from jax.experimental.pallas import tpu as pltpu
from jax.experimental.pallas import tpu_sc as plsc

N = 10000
E = 320000
H = 128
G = 512
NLAYERS = 3

NC = 2
NS = 16
CHUNK = 64
NWORK = NC * NS
CPW = 160
NBUF = 2
EP = NWORK * CPW * CHUNK
NPAD = 10112
ROWS_PT = NPAD // NS
OUT_MAIN = 632
OUT_TAIL = N - (NS - 1) * OUT_MAIN

_MESH = plsc.VectorSubcoreMesh(
    core_axis_name="c", subcore_axis_name="s", num_cores=NC, num_subcores=NS
)


@functools.partial(
    pl.kernel,
    out_type=jax.ShapeDtypeStruct((NC, N, H), jnp.float32),
    mesh=_MESH,
    scratch_types=[
        pltpu.VMEM_SHARED((NPAD, H), jnp.float32),
        pltpu.VMEM((CPW, CHUNK), jnp.int32),
        pltpu.VMEM((CHUNK, H), jnp.float32),
    ],
)
def _deg_kernel(dstm_hbm, zeros_hbm, ones_hbm, out_hbm, deg_sh, dstv, ones_v):
    c = lax.axis_index("c")
    t = lax.axis_index("s")
    w = c * NS + t
    pltpu.sync_copy(dstm_hbm.at[pl.ds(w * CPW, CPW)], dstv)
    pltpu.sync_copy(ones_hbm, ones_v)
    pltpu.sync_copy(
        zeros_hbm.at[pl.ds(t * ROWS_PT, ROWS_PT)],
        deg_sh.at[pl.ds(t * ROWS_PT, ROWS_PT)],
    )
    plsc.subcore_barrier()

    def body(j, carry):
        pltpu.sync_copy(ones_v, deg_sh.at[dstv.at[j]], add=True)
        return carry

    lax.fori_loop(0, CPW, body, 0)
    plsc.subcore_barrier()

    @pl.when(t < NS - 1)
    def _copy_main():
        pltpu.sync_copy(
            deg_sh.at[pl.ds(t * OUT_MAIN, OUT_MAIN)],
            out_hbm.at[c, pl.ds(t * OUT_MAIN, OUT_MAIN)],
        )

    @pl.when(t == NS - 1)
    def _copy_tail():
        pltpu.sync_copy(
            deg_sh.at[pl.ds((NS - 1) * OUT_MAIN, OUT_TAIL)],
            out_hbm.at[c, pl.ds((NS - 1) * OUT_MAIN, OUT_TAIL)],
        )


@functools.partial(
    pl.kernel,
    out_type=jax.ShapeDtypeStruct((NC, N, H), jnp.float32),
    mesh=_MESH,
    scratch_types=[
        pltpu.VMEM_SHARED((NPAD, H), jnp.float32),
        pltpu.VMEM((CPW * CHUNK,), jnp.int32),
        pltpu.VMEM((CPW, CHUNK), jnp.int32),
        pltpu.VMEM((CHUNK, H), jnp.float32),
        pltpu.VMEM((CHUNK, H), jnp.float32),
        pltpu.SemaphoreType.DMA,
        pltpu.SemaphoreType.DMA,
    ],
)
def _mp_kernel(hp_hbm, srcf_hbm, dstm_hbm, zeros_hbm, out_hbm,
               acc_sh, srcv, dstv, rows0, rows1, semA, semB):
    c = lax.axis_index("c")
    t = lax.axis_index("s")
    w = c * NS + t
    epw = CPW * CHUNK
    pltpu.sync_copy(srcf_hbm.at[pl.ds(w * epw, epw)], srcv)
    pltpu.sync_copy(dstm_hbm.at[pl.ds(w * CPW, CPW)], dstv)
    pltpu.sync_copy(
        zeros_hbm.at[pl.ds(t * ROWS_PT, ROWS_PT)],
        acc_sh.at[pl.ds(t * ROWS_PT, ROWS_PT)],
    )
    plsc.subcore_barrier()

    bufs = (rows0, rows1)
    sems = (semA, semB)

    def sidx(j):
        return srcv.at[pl.ds(pl.multiple_of(j * CHUNK, CHUNK), CHUNK)]

    for b in range(NBUF):
        pltpu.async_copy(hp_hbm.at[sidx(b)], bufs[b], sems[b])

    def body(k, carry):
        for b in range(NBUF):
            j = k * NBUF + b
            pltpu.make_async_copy(hp_hbm.at[sidx(j)], bufs[b], sems[b]).wait()
            pltpu.sync_copy(bufs[b], acc_sh.at[dstv.at[j]], add=True)

            @pl.when(j + NBUF < CPW)
            def _(b=b, j=j):
                pltpu.async_copy(hp_hbm.at[sidx(j + NBUF)], bufs[b], sems[b])

        return carry

    lax.fori_loop(0, CPW // NBUF, body, 0)
    plsc.subcore_barrier()

    @pl.when(t < NS - 1)
    def _copy_main():
        pltpu.sync_copy(
            acc_sh.at[pl.ds(t * OUT_MAIN, OUT_MAIN)],
            out_hbm.at[c, pl.ds(t * OUT_MAIN, OUT_MAIN)],
        )

    @pl.when(t == NS - 1)
    def _copy_tail():
        pltpu.sync_copy(
            acc_sh.at[pl.ds((NS - 1) * OUT_MAIN, OUT_TAIL)],
            out_hbm.at[c, pl.ds((NS - 1) * OUT_MAIN, OUT_TAIL)],
        )


def _embed_body(tok_ref, emb_ref, w_ref, degp_ref, hp_ref, dinv_ref):
    tok = tok_ref[...]
    oh = (tok == lax.broadcasted_iota(jnp.int32, (N, 16), 1)).astype(jnp.float32)
    ew = jnp.dot(emb_ref[...], w_ref[...], preferred_element_type=jnp.float32)
    x0w = jnp.dot(oh, ew, preferred_element_type=jnp.float32)
    degp = degp_ref[...]
    deg = degp[0, :, 0:1] + degp[1, :, 0:1] + 1.0
    dinv = lax.rsqrt(deg)
    dinv_ref[...] = jnp.broadcast_to(dinv, (N, 8))
    hp_ref[...] = x0w * dinv


def _mid_body(accp_ref, hp_ref, dinv_ref, b_ref, g_ref, be_ref, w_ref, out_ref):
    acc = accp_ref[0] + accp_ref[1]
    hp = hp_ref[...]
    dinv = dinv_ref[:, 0:1]
    agg = dinv * (acc + hp) + b_ref[...]
    m = jnp.mean(agg, axis=0, keepdims=True)
    xc = agg - m
    v = jnp.mean(xc * xc, axis=0, keepdims=True)
    xn = xc * lax.rsqrt(v + 1e-5)
    x = jnp.maximum(g_ref[...] * xn + be_ref[...], 0.0)
    out_ref[...] = jnp.dot(x, w_ref[...], preferred_element_type=jnp.float32) * dinv


def _final_body(accp_ref, hp_ref, dinv_ref, b_ref, g_ref, be_ref,
                br_ref, wp_ref, bp_ref, out_ref):
    acc = accp_ref[0] + accp_ref[1]
    hp = hp_ref[...]
    dinv = dinv_ref[:, 0:1]
    agg = dinv * (acc + hp) + b_ref[...]
    m = jnp.mean(agg, axis=0, keepdims=True)
    xc = agg - m
    v = jnp.mean(xc * xc, axis=0, keepdims=True)
    xn = xc * lax.rsqrt(v + 1e-5)
    x = jnp.maximum(g_ref[...] * xn + be_ref[...], 0.0)

    nchunk = 2000
    pooled = jnp.zeros((G, H), jnp.float32)
    cnt = jnp.zeros((G, 1), jnp.float32)
    giota = lax.broadcasted_iota(jnp.int32, (G, nchunk), 0)
    for kc in range(N // nchunk):
        bid = br_ref[0:1, kc * nchunk:(kc + 1) * nchunk]
        ohT = (giota == bid).astype(jnp.float32)
        pooled = pooled + jnp.dot(
            ohT, x[kc * nchunk:(kc + 1) * nchunk],
            preferred_element_type=jnp.float32)
        cnt = cnt + jnp.sum(ohT, axis=1, keepdims=True)
    pooled = pooled / jnp.maximum(cnt, 1.0)
    out_ref[...] = (
        jnp.dot(pooled, wp_ref[...], preferred_element_type=jnp.float32)
        + bp_ref[...]
    )


def _tc(body, out_shape, *args):
    return pl.pallas_call(body, out_shape=out_shape)(*args)


def kernel(node_tokens, edge_index, batch_ids, emb, Ws, bs, gammas, betas, Wp, bp):
    src = edge_index[0].astype(jnp.int32)
    dst = edge_index[1].astype(jnp.int32)
    pad = EP - E
    srcf = jnp.concatenate([src, jnp.zeros((pad,), jnp.int32)])
    dstm = jnp.concatenate([dst, jnp.full((pad,), N, jnp.int32)]).reshape(EP // CHUNK, CHUNK)

    zerosH = jnp.zeros((NPAD, H), jnp.float32)
    onesH = jnp.ones((CHUNK, H), jnp.float32)

    degp = _deg_kernel(dstm, zerosH, onesH)

    tok2 = node_tokens.astype(jnp.int32).reshape(N, 1)
    embp = jnp.pad(emb, ((0, 16 - emb.shape[0]), (0, 0)))

    hp, dinv8 = _tc(_embed_body,
                    (jax.ShapeDtypeStruct((N, H), jnp.float32),
                     jax.ShapeDtypeStruct((N, 8), jnp.float32)),
                    tok2, embp, Ws[0], degp)

    for l in range(NLAYERS - 1):
        accp = _mp_kernel(hp, srcf, dstm, zerosH)
        hp = _tc(_mid_body, jax.ShapeDtypeStruct((N, H), jnp.float32),
                 accp, hp, dinv8,
                 bs[l].reshape(1, H), gammas[l].reshape(1, H),
                 betas[l].reshape(1, H), Ws[l + 1])

    accp = _mp_kernel(hp, srcf, dstm, zerosH)
    batch_row = batch_ids.astype(jnp.int32).reshape(1, N)
    l = NLAYERS - 1
    out = _tc(_final_body, jax.ShapeDtypeStruct((G, H), jnp.float32),
              accp, hp, dinv8,
              bs[l].reshape(1, H), gammas[l].reshape(1, H),
              betas[l].reshape(1, H), batch_row, Wp, bp.reshape(1, H))
    return out
